# baseline (speedup 1.0000x reference)
import jax
import jax.numpy as jnp
from jax import lax
from jax.experimental import pallas as pl
from jax.experimental.pallas import tpu as pltpu
from jax.experimental.pallas import tpu_sc as plsc

B = 16384
D = 32
L = 16
NC = 2
NS = 16
NW = NC * NS
BPW = B // NW
CHUNK = 128
NCHUNK = BPW // CHUNK
GROUPS = BPW // L


def _sc_body(uid_hbm, iid_hbm, ut_hbm, it_hbm, out_hbm,
             uid_v, iid_v, u_buf, i_buf, out_v, sem_u, sem_i):
    wid = lax.axis_index("s") * NC + lax.axis_index("c")
    base = wid * BPW

    pltpu.sync_copy(uid_hbm.at[pl.ds(base, BPW)], uid_v)
    pltpu.sync_copy(iid_hbm.at[pl.ds(base, BPW)], iid_v)

    copies = []
    for d in range(D):
        for c in range(NCHUNK):
            idx_u = uid_v.at[pl.ds(c * CHUNK, CHUNK)]
            idx_i = iid_v.at[pl.ds(c * CHUNK, CHUNK)]
            dst = pl.ds(d * BPW + c * CHUNK, CHUNK)
            copies.append(
                pltpu.async_copy(ut_hbm.at[d].at[idx_u], u_buf.at[dst], sem_u))
            copies.append(
                pltpu.async_copy(it_hbm.at[d].at[idx_i], i_buf.at[dst], sem_i))
    for cp in copies:
        cp.wait()

    def group(g, _):
        sl = g * L
        acc = jnp.zeros((L,), jnp.float32)
        for d in range(D):
            acc = acc + u_buf[pl.ds(d * BPW + sl, L)] * i_buf[pl.ds(d * BPW + sl, L)]
        out_v[pl.ds(sl, L)] = acc
        return 0

    lax.fori_loop(0, GROUPS, group, 0)

    pltpu.sync_copy(out_v, out_hbm.at[pl.ds(base, BPW)])


def kernel(user_ids, item_ids, user_table, item_table):
    ut = user_table.T
    it = item_table.T
    mesh = plsc.VectorSubcoreMesh(core_axis_name="c", subcore_axis_name="s")
    f = pl.kernel(
        _sc_body,
        mesh=mesh,
        compiler_params=pltpu.CompilerParams(
            use_tc_tiling_on_sc=False, needs_layout_passes=False),
        out_type=jax.ShapeDtypeStruct((B,), jnp.float32),
        scratch_types=[
            pltpu.VMEM((BPW,), jnp.int32),
            pltpu.VMEM((BPW,), jnp.int32),
            pltpu.VMEM((D * BPW,), jnp.float32),
            pltpu.VMEM((D * BPW,), jnp.float32),
            pltpu.VMEM((BPW,), jnp.float32),
            pltpu.SemaphoreType.DMA,
            pltpu.SemaphoreType.DMA,
        ],
    )
    return f(user_ids.astype(jnp.int32), item_ids.astype(jnp.int32), ut, it)


# speedup vs baseline: 19.9029x; 19.9029x over previous
import jax
import jax.numpy as jnp
from jax import lax
from jax.experimental import pallas as pl
from jax.experimental.pallas import tpu as pltpu
from jax.experimental.pallas import tpu_sc as plsc

B = 16384
D = 32
L = 16
NC = 2
NS = 16
NW = NC * NS
BPW = B // NW
LANE = 128
SLOTS = 8


def _sc_body(uid_hbm, iid_hbm, ut_hbm, it_hbm, out_hbm,
             uid_v, iid_v, ustage, istage, out_v, tmp_v, sem_u, sem_i):
    wid = lax.axis_index("s") * NC + lax.axis_index("c")
    base = wid * BPW

    pltpu.sync_copy(uid_hbm.at[pl.ds(base, BPW)], uid_v)
    pltpu.sync_copy(iid_hbm.at[pl.ds(base, BPW)], iid_v)

    lanes = lax.iota(jnp.int32, L)
    active_lo = lanes < SLOTS
    zeros = jnp.zeros((L,), jnp.float32)

    def group(g, _):
        uvec = uid_v[pl.ds(g * L, L)]
        ivec = iid_v[pl.ds(g * L, L)]
        url = uvec & (LANE - 1)
        irl = ivec & (LANE - 1)
        svec = lanes & (SLOTS - 1)
        halves = []
        for half in range(2):
            cps = []
            for k in range(SLOTS):
                p = half * SLOTS + k
                ru = pl.multiple_of((uvec[p] >> 7) * LANE, LANE)
                ri = pl.multiple_of((ivec[p] >> 7) * LANE, LANE)
                cps.append(pltpu.async_copy(
                    ut_hbm.at[:, pl.ds(ru, LANE)], ustage.at[k], sem_u))
                cps.append(pltpu.async_copy(
                    it_hbm.at[:, pl.ds(ri, LANE)], istage.at[k], sem_i))
            for cp in cps:
                cp.wait()
            if half == 0:
                rlu, rli = url, irl
            else:
                rlu = plsc.load_gather(uid_v, [g * L + SLOTS + svec]) & (LANE - 1)
                rli = plsc.load_gather(iid_v, [g * L + SLOTS + svec]) & (LANE - 1)
            acc = zeros
            for d in range(D):
                dvec = jnp.full((L,), d, jnp.int32)
                ug = plsc.load_gather(ustage, [svec, dvec, rlu])
                vg = plsc.load_gather(istage, [svec, dvec, rli])
                acc = acc + ug * vg
            halves.append(acc)
        tmp_v[...] = halves[1]
        shifted = plsc.load_gather(tmp_v, [svec])
        out_v[pl.ds(g * L, L)] = jnp.where(active_lo, halves[0], shifted)
        return 0

    lax.fori_loop(0, BPW // L, group, 0)
    pltpu.sync_copy(out_v, out_hbm.at[pl.ds(base, BPW)])


def kernel(user_ids, item_ids, user_table, item_table):
    ut = user_table.T
    it = item_table.T
    mesh = plsc.VectorSubcoreMesh(core_axis_name="c", subcore_axis_name="s")
    f = pl.kernel(
        _sc_body,
        mesh=mesh,
        compiler_params=pltpu.CompilerParams(
            use_tc_tiling_on_sc=True, needs_layout_passes=False),
        out_type=jax.ShapeDtypeStruct((B,), jnp.float32),
        scratch_types=[
            pltpu.VMEM((BPW,), jnp.int32),
            pltpu.VMEM((BPW,), jnp.int32),
            pltpu.VMEM((SLOTS, D, LANE), jnp.float32),
            pltpu.VMEM((SLOTS, D, LANE), jnp.float32),
            pltpu.VMEM((BPW,), jnp.float32),
            pltpu.VMEM((L,), jnp.float32),
            pltpu.SemaphoreType.DMA,
            pltpu.SemaphoreType.DMA,
        ],
    )
    return f(user_ids.astype(jnp.int32), item_ids.astype(jnp.int32), ut, it)
